# baseline (speedup 1.0000x reference)
import jax
import jax.numpy as jnp
from jax import lax
from jax.experimental import pallas as pl
from jax.experimental.pallas import tpu as pltpu
from jax.experimental.pallas import tpu_sc as plsc

_PATCH = 14
_NC, _NS = 2, 16
_NW = _NC * _NS
_LANES = 16


def _patch_perms(B, L):
    key = jax.random.key(42)
    keys = jax.random.split(key, B)
    return jnp.stack([jax.random.permutation(k, L) for k in keys])


def kernel(x):
    B, C, H, W = x.shape
    p = _PATCH
    nh, nw = H // p, W // p
    L = nh * nw
    rows = H * nw
    n_img = B * C
    assert n_img % _NW == 0
    imgs_per_w = n_img // _NW
    assert C % imgs_per_w == 0
    assert rows % _LANES == 0

    perms = _patch_perms(B, L)
    sh = (perms // nw).reshape(B, nh, nw)
    sw = (perms % nw).reshape(B, nh, nw)
    r = jnp.arange(p, dtype=jnp.int32)
    off = (sh[:, :, None, :] * p + r[None, None, :, None]) * nw \
        + sw[:, :, None, :]
    off = off.reshape(B, rows).astype(jnp.int32)

    x2 = x.reshape(n_img * rows, p)

    def body(x_ref, off_ref, out_ref,
             buf0, buf1, idx0, idx1, offv,
             sg0, sg1, swr0, swr1):
        cid = lax.axis_index("c")
        sid = lax.axis_index("s")
        wid = sid * _NC + cid
        first = wid * imgs_per_w
        b = first // C
        pltpu.sync_copy(off_ref.at[b], offv)

        def make_idx(idx_ref, base):
            basev = jnp.full((_LANES,), base, dtype=jnp.int32)

            def step(t, carry):
                sl = pl.ds(t * _LANES, _LANES)
                idx_ref[sl] = offv[sl] + basev
                return carry

            lax.fori_loop(0, rows // _LANES, step, 0)

        def pair(k, carry):
            base0 = (first + 2 * k) * rows
            base1 = base0 + rows
            make_idx(idx0, base0)
            g0 = pltpu.async_copy(x_ref.at[idx0], buf0, sg0)
            make_idx(idx1, base1)
            g1 = pltpu.async_copy(x_ref.at[idx1], buf1, sg1)
            g0.wait()
            w0 = pltpu.async_copy(buf0, out_ref.at[pl.ds(base0, rows)], swr0)
            g1.wait()
            w1 = pltpu.async_copy(buf1, out_ref.at[pl.ds(base1, rows)], swr1)
            w0.wait()
            w1.wait()
            return carry

        lax.fori_loop(0, imgs_per_w // 2, pair, 0)

    f = pl.kernel(
        body,
        out_type=jax.ShapeDtypeStruct((n_img * rows, p), jnp.float32),
        mesh=plsc.VectorSubcoreMesh(
            core_axis_name="c", subcore_axis_name="s",
            num_cores=_NC, num_subcores=_NS),
        compiler_params=pltpu.CompilerParams(use_tc_tiling_on_sc=False),
        scratch_types=[
            pltpu.VMEM((rows, p), jnp.float32),
            pltpu.VMEM((rows, p), jnp.float32),
            pltpu.VMEM((rows,), jnp.int32),
            pltpu.VMEM((rows,), jnp.int32),
            pltpu.VMEM((rows,), jnp.int32),
            pltpu.SemaphoreType.DMA,
            pltpu.SemaphoreType.DMA,
            pltpu.SemaphoreType.DMA,
            pltpu.SemaphoreType.DMA,
        ],
    )
    out2 = f(x2, off)
    return out2.reshape(B, C, H, W)


# speedup vs baseline: 5.2829x; 5.2829x over previous
import jax
import jax.numpy as jnp
from jax import lax
from jax.experimental import pallas as pl
from jax.experimental.pallas import tpu as pltpu
from jax.experimental.pallas import tpu_sc as plsc

_PATCH = 14
_NC, _NS = 2, 16
_NW = _NC * _NS
_LANES = 16
_CHUNKS = 8


def _patch_perms(B, L):
    key = jax.random.key(42)
    keys = jax.random.split(key, B)
    return jnp.stack([jax.random.permutation(k, L) for k in keys])


def kernel(x):
    B, C, H, W = x.shape
    p = _PATCH
    nh, nw = H // p, W // p
    L = nh * nw
    rows = H * nw
    img = H * W
    n_img = B * C
    assert n_img % _NW == 0
    imgs_per_w = n_img // _NW
    assert imgs_per_w % 2 == 0
    assert C % imgs_per_w == 0
    assert rows % (_LANES * _CHUNKS) == 0

    blk_per_chunk = rows // (_LANES * _CHUNKS)
    chunk_elems = img // _CHUNKS

    perms = _patch_perms(B, L)
    sh = (perms // nw).reshape(B, nh, nw)
    sw = (perms % nw).reshape(B, nh, nw)
    r = jnp.arange(p, dtype=jnp.int32)
    off = (sh[:, :, None, :] * p + r[None, None, :, None]) * nw \
        + sw[:, :, None, :]
    off = off.reshape(B, rows).astype(jnp.int32)

    x2 = x.reshape(n_img * img)

    def body(x_ref, off_ref, out_ref,
             in0, in1, st0, st1, offv, si0, si1, ss0, ss1):
        cid = lax.axis_index("c")
        sid = lax.axis_index("s")
        wid = sid * _NC + cid
        first = wid * imgs_per_w
        b = first // C
        pltpu.sync_copy(off_ref.at[b], offv)

        iota = lax.iota(jnp.int32, _LANES)
        i14 = iota * jnp.full((_LANES,), p, jnp.int32)
        one = jnp.full((_LANES,), 1, jnp.int32)
        fourteen = jnp.full((_LANES,), p, jnp.int32)

        ins = [in0, in1]
        isems = [si0, si1]
        stages = [st0, st1]
        ssems = [ss0, ss1]

        def shuffle_chunk(inbuf, stage, row0):
            def blk(tt, carry):
                offc = offv[pl.ds(row0 + tt * _LANES, _LANES)]
                inb = offc * fourteen
                outb = i14 + jnp.full((_LANES,), tt * (_LANES * p),
                                      jnp.int32)
                for _ in range(p):
                    v = plsc.load_gather(inbuf, [inb])
                    plsc.store_scatter(stage, [outb], v)
                    inb = inb + one
                    outb = outb + one
                return carry

            lax.fori_loop(0, blk_per_chunk, blk, 0)

        pltpu.async_copy(x_ref.at[pl.ds(first * img, img)], in0, si0)
        pltpu.async_copy(x_ref.at[pl.ds((first + 1) * img, img)], in1, si1)

        def pair(k2, carry):
            for h in range(2):
                k = k2 * 2 + h
                gbase = (first + k) * img
                pltpu.make_async_copy(
                    x_ref.at[pl.ds(gbase, img)], ins[h], isems[h]).wait()
                for c in range(_CHUNKS):
                    st = stages[c % 2]
                    if c >= 2:
                        pltpu.make_async_copy(
                            st,
                            out_ref.at[pl.ds(
                                gbase + (c - 2) * chunk_elems,
                                chunk_elems)],
                            ssems[c % 2]).wait()
                    shuffle_chunk(ins[h], st, c * (_LANES * blk_per_chunk))
                    pltpu.async_copy(
                        st,
                        out_ref.at[pl.ds(gbase + c * chunk_elems,
                                         chunk_elems)],
                        ssems[c % 2])
                @pl.when(k2 + 1 < imgs_per_w // 2)
                def _():
                    pltpu.async_copy(
                        x_ref.at[pl.ds(gbase + 2 * img, img)],
                        ins[h], isems[h])
                for c in (_CHUNKS - 2, _CHUNKS - 1):
                    pltpu.make_async_copy(
                        stages[c % 2],
                        out_ref.at[pl.ds(gbase + c * chunk_elems,
                                         chunk_elems)],
                        ssems[c % 2]).wait()
            return carry

        lax.fori_loop(0, imgs_per_w // 2, pair, 0)

    f = pl.kernel(
        body,
        out_type=jax.ShapeDtypeStruct((n_img * img,), jnp.float32),
        mesh=plsc.VectorSubcoreMesh(
            core_axis_name="c", subcore_axis_name="s",
            num_cores=_NC, num_subcores=_NS),
        compiler_params=pltpu.CompilerParams(
            use_tc_tiling_on_sc=False, needs_layout_passes=False),
        scratch_types=[
            pltpu.VMEM((img,), jnp.float32),
            pltpu.VMEM((img,), jnp.float32),
            pltpu.VMEM((chunk_elems,), jnp.float32),
            pltpu.VMEM((chunk_elems,), jnp.float32),
            pltpu.VMEM((rows,), jnp.int32),
            pltpu.SemaphoreType.DMA,
            pltpu.SemaphoreType.DMA,
            pltpu.SemaphoreType.DMA,
            pltpu.SemaphoreType.DMA,
        ],
    )
    out2 = f(x2, off)
    return out2.reshape(B, C, H, W)


# speedup vs baseline: 5.8054x; 1.0989x over previous
import functools

import jax
import jax.numpy as jnp
import numpy as np
from jax import lax
from jax.experimental import pallas as pl
from jax.experimental.pallas import tpu as pltpu
from jax.experimental.pallas import tpu_sc as plsc

_PATCH = 14
_NC, _NS = 2, 16
_NW = _NC * _NS
_LANES = 16
_CHUNKS = 8

_TABLE_CACHE = {}


def _perm_tables(B, nh, nw):
    key_ = (B, nh, nw)
    p = _PATCH
    L = nh * nw
    rows = nh * p * nw

    def make():
        key = jax.random.key(42)
        keys = jax.random.split(key, B)
        return jnp.stack([jax.random.permutation(k, L) for k in keys])

    if key_ not in _TABLE_CACHE:
        try:
            with jax.default_device(jax.devices("cpu")[0]), \
                    jax.ensure_compile_time_eval():
                _TABLE_CACHE[key_] = np.asarray(make())
        except Exception:
            pass

    if key_ in _TABLE_CACHE:
        perms = _TABLE_CACHE[key_]
        xp = np
    else:
        perms = make()
        xp = jnp

    sh = (perms // nw).reshape(B, nh, nw)
    sw = (perms % nw).reshape(B, nh, nw)
    r = xp.arange(p, dtype=xp.int32)
    off_h = sh[:, :, None, :] * p + r[None, None, :, None]
    off_w = xp.broadcast_to(sw[:, :, None, :] * p, (B, nh, p, nw))
    return (off_h.reshape(B, rows).astype(xp.int32),
            off_w.reshape(B, rows).astype(xp.int32))


def kernel(x):
    B, C, H, W = x.shape
    p = _PATCH
    nh, nw = H // p, W // p
    rows = H * nw
    n_img = B * C
    assert n_img % _NW == 0
    imgs_per_w = n_img // _NW
    assert imgs_per_w % 2 == 0
    assert C % imgs_per_w == 0
    assert H % _CHUNKS == 0

    chunk_h = H // _CHUNKS
    blk_per_chunk = chunk_h

    off_h_np, off_w_np = _perm_tables(B, nh, nw)
    off_h = jnp.asarray(off_h_np)
    off_w = jnp.asarray(off_w_np)

    x3 = x.reshape(n_img, H, W)

    def body(x_ref, offh_ref, offw_ref, out_ref,
             in0, in1, st0, st1, offh_v, offw_v, si0, si1, ss0, ss1):
        cid = lax.axis_index("c")
        sid = lax.axis_index("s")
        wid = sid * _NC + cid
        first = wid * imgs_per_w
        b = first // C
        pltpu.sync_copy(offh_ref.at[b], offh_v)
        pltpu.sync_copy(offw_ref.at[b], offw_v)

        iota = lax.iota(jnp.int32, _LANES)
        i14 = iota * jnp.full((_LANES,), p, jnp.int32)
        one = jnp.full((_LANES,), 1, jnp.int32)
        cvecs = [i14 + jnp.full((_LANES,), j, jnp.int32) for j in range(p)]

        ins = [in0, in1]
        isems = [si0, si1]
        stages = [st0, st1]
        ssems = [ss0, ss1]

        def shuffle_chunk(inbuf, stage, c):
            def blk(tt, carry):
                o0 = (c * chunk_h + tt) * nw
                hvec = offh_v[pl.ds(o0, _LANES)]
                wvec = offw_v[pl.ds(o0, _LANES)]
                row = stage.at[tt]
                for j in range(p):
                    v = plsc.load_gather(inbuf, [hvec, wvec])
                    plsc.store_scatter(row, [cvecs[j]], v)
                    wvec = wvec + one
                return carry

            lax.fori_loop(0, blk_per_chunk, blk, 0)

        pltpu.async_copy(x_ref.at[first], in0, si0)
        pltpu.async_copy(x_ref.at[first + 1], in1, si1)

        def pair(k2, carry):
            for h in range(2):
                k = k2 * 2 + h
                img = first + k
                pltpu.make_async_copy(
                    x_ref.at[img], ins[h], isems[h]).wait()
                for c in range(_CHUNKS):
                    st = stages[c % 2]
                    if c >= 2:
                        pltpu.make_async_copy(
                            st,
                            out_ref.at[img, pl.ds((c - 2) * chunk_h,
                                                  chunk_h)],
                            ssems[c % 2]).wait()
                    shuffle_chunk(ins[h], st, c)
                    pltpu.async_copy(
                        st,
                        out_ref.at[img, pl.ds(c * chunk_h, chunk_h)],
                        ssems[c % 2])
                @pl.when(k2 + 1 < imgs_per_w // 2)
                def _():
                    pltpu.async_copy(x_ref.at[img + 2], ins[h], isems[h])
                for c in (_CHUNKS - 2, _CHUNKS - 1):
                    pltpu.make_async_copy(
                        stages[c % 2],
                        out_ref.at[img, pl.ds(c * chunk_h, chunk_h)],
                        ssems[c % 2]).wait()
            return carry

        lax.fori_loop(0, imgs_per_w // 2, pair, 0)

    f = pl.kernel(
        body,
        out_type=jax.ShapeDtypeStruct((n_img, H, W), jnp.float32),
        mesh=plsc.VectorSubcoreMesh(
            core_axis_name="c", subcore_axis_name="s",
            num_cores=_NC, num_subcores=_NS),
        compiler_params=pltpu.CompilerParams(
            use_tc_tiling_on_sc=False, needs_layout_passes=False),
        scratch_types=[
            pltpu.VMEM((H, W), jnp.float32),
            pltpu.VMEM((H, W), jnp.float32),
            pltpu.VMEM((chunk_h, W), jnp.float32),
            pltpu.VMEM((chunk_h, W), jnp.float32),
            pltpu.VMEM((rows,), jnp.int32),
            pltpu.VMEM((rows,), jnp.int32),
            pltpu.SemaphoreType.DMA,
            pltpu.SemaphoreType.DMA,
            pltpu.SemaphoreType.DMA,
            pltpu.SemaphoreType.DMA,
        ],
    )
    out3 = f(x3, off_h, off_w)
    return out3.reshape(B, C, H, W)
